# Initial kernel scaffold; baseline (speedup 1.0000x reference)
import jax
import jax.numpy as jnp
from jax.experimental import pallas as pl


def kernel(points, cp, voxel, plane, quat):
    raise NotImplementedError("write your pallas kernel here")


# speedup vs baseline: 3.4651x; 3.4651x over previous
import functools

import jax
import jax.numpy as jnp
from jax import lax
from jax.experimental import pallas as pl
from jax.experimental.pallas import tpu as pltpu
from jax.experimental.pallas import tpu_sc as plsc

GRID = 32
NCELL = GRID ** 3
NPTS = 8192
NBATCH = 16
L = 16
NGROUPS = NPTS // L
NWORKERS = 32


def _affine_params(plane, quat):
    eye = jnp.eye(3, dtype=jnp.float32)
    n = plane[:, :3]
    n = n / (jnp.linalg.norm(n, axis=1, keepdims=True) + 1e-8)
    d = plane[:, 3:4]
    mp = eye[None] - 2.0 * n[:, :, None] * n[:, None, :]
    op = -2.0 * d * n
    q = quat / (jnp.linalg.norm(quat, axis=1, keepdims=True) + 1e-8)
    w = q[:, 0]
    qv = q[:, 1:]
    s2 = jnp.sum(qv * qv, axis=1)
    zero = jnp.zeros_like(w)
    kx, ky, kz = qv[:, 0], qv[:, 1], qv[:, 2]
    skew = jnp.stack([
        jnp.stack([zero, -kz, ky], axis=1),
        jnp.stack([kz, zero, -kx], axis=1),
        jnp.stack([-ky, kx, zero], axis=1),
    ], axis=1)
    mq = ((1.0 - 2.0 * s2)[:, None, None] * eye[None]
          + 2.0 * qv[:, :, None] * qv[:, None, :]
          + 2.0 * w[:, None, None] * skew)
    oq = jnp.zeros((quat.shape[0], 3), jnp.float32)
    mats = jnp.concatenate([mp, mq], axis=0).reshape(4, 9)
    offs = jnp.concatenate([op, oq], axis=0)
    return jnp.concatenate([mats, offs], axis=1)


@functools.cache
def _make_kernel():
    mesh = plsc.VectorSubcoreMesh(core_axis_name="c", subcore_axis_name="s")

    @functools.partial(
        pl.kernel,
        mesh=mesh,
        out_type=jax.ShapeDtypeStruct((2, NWORKERS, L), jnp.float32),
        scratch_types=[
            pltpu.VMEM((3, NPTS), jnp.float32),
            pltpu.VMEM((3, NPTS), jnp.float32),
            pltpu.VMEM((NPTS,), jnp.int32),
            pltpu.VMEM((NPTS,), jnp.float32),
            pltpu.VMEM((NPTS,), jnp.float32),
            pltpu.VMEM((NPTS,), jnp.float32),
            pltpu.VMEM((NPTS,), jnp.float32),
            pltpu.VMEM((4, 12, L), jnp.float32),
            pltpu.VMEM((L,), jnp.float32),
            pltpu.SemaphoreType.DMA,
        ],
    )
    def sym_loss_kernel(ptsx_hbm, tabx_hbm, taby_hbm, tabz_hbm, tabm_hbm,
                        par_hbm, out_hbm,
                        pts_v, tp_v, idx_v,
                        g0_v, g1_v, g2_v, g3_v, par_v, stage_v, sem):
        c = lax.axis_index("c")
        s = lax.axis_index("s")
        wid = s * 2 + c
        t = wid // NBATCH
        b = wid % NBATCH
        pltpu.sync_copy(ptsx_hbm.at[b], pts_v)
        pltpu.sync_copy(par_hbm, par_v)
        base = b * NCELL

        for g in range(2):
            row = 2 * g + t
            m = [par_v[row, k, :] for k in range(12)]

            def body_idx(i, carry):
                off = i * L
                x = pts_v[0, pl.ds(off, L)]
                y = pts_v[1, pl.ds(off, L)]
                z = pts_v[2, pl.ds(off, L)]
                tx = m[0] * x + m[1] * y + m[2] * z + m[9]
                ty = m[3] * x + m[4] * y + m[5] * z + m[10]
                tz = m[6] * x + m[7] * y + m[8] * z + m[11]
                tp_v[0, pl.ds(off, L)] = tx
                tp_v[1, pl.ds(off, L)] = ty
                tp_v[2, pl.ds(off, L)] = tz
                ux = jnp.clip((tx + 0.5 - 0.015625) * 32.0, 0.0, 31.0)
                uy = jnp.clip((ty + 0.5 - 0.015625) * 32.0, 0.0, 31.0)
                uz = jnp.clip((tz + 0.5 - 0.015625) * 32.0, 0.0, 31.0)
                ix = (ux + 0.5).astype(jnp.int32)
                iy = (uy + 0.5).astype(jnp.int32)
                iz = (uz + 0.5).astype(jnp.int32)
                flat = ix * (GRID * GRID) + iy * GRID + iz
                idx_v[pl.ds(off, L)] = flat + base
                return carry

            lax.fori_loop(0, NGROUPS, body_idx, 0)

            cp0 = pltpu.async_copy(tabx_hbm.at[idx_v], g0_v, sem)
            cp1 = pltpu.async_copy(taby_hbm.at[idx_v], g1_v, sem)
            cp2 = pltpu.async_copy(tabz_hbm.at[idx_v], g2_v, sem)
            cp3 = pltpu.async_copy(tabm_hbm.at[idx_v], g3_v, sem)
            cp0.wait()
            cp1.wait()
            cp2.wait()
            cp3.wait()

            def body_acc(i, acc):
                off = i * L
                dx = tp_v[0, pl.ds(off, L)] - g0_v[pl.ds(off, L)]
                dy = tp_v[1, pl.ds(off, L)] - g1_v[pl.ds(off, L)]
                dz = tp_v[2, pl.ds(off, L)] - g2_v[pl.ds(off, L)]
                msk = g3_v[pl.ds(off, L)]
                return acc + (dx * dx + dy * dy + dz * dz) * msk

            acc = lax.fori_loop(0, NGROUPS, body_acc,
                                jnp.zeros((L,), jnp.float32))
            stage_v[...] = acc
            pltpu.sync_copy(stage_v, out_hbm.at[g, wid])

    return sym_loss_kernel


def kernel(points, cp, voxel, plane, quat):
    pts_t = points.transpose(0, 2, 1)
    tabm = 1.0 - voxel.reshape(NBATCH * NCELL)
    cp_t = cp.transpose(2, 0, 1).reshape(3, NBATCH * NCELL)
    par = jnp.broadcast_to(_affine_params(plane, quat)[:, :, None],
                           (4, 12, L))
    parts = _make_kernel()(pts_t, cp_t[0], cp_t[1], cp_t[2], tabm,
                           par)
    ref_loss = jnp.sum(parts[0]) / NBATCH
    rot_loss = jnp.sum(parts[1]) / NBATCH
    return (ref_loss, rot_loss)
